# baseline (speedup 1.0000x reference)
import functools

import jax
import jax.numpy as jnp
from jax import lax
from jax.experimental import pallas as pl
from jax.experimental.pallas import tpu as pltpu
from jax.experimental.pallas import tpu_sc as plsc

N = 10000
E = 320000
D = 128
DE = 4
HID = 128
NFINE = 10

NC = 2
NS = 16
NW = NC * NS
EPW = E // NW
CH = 80
NCHUNK = EPW // CH
NPAD = 10240
RPS = NPAD // NS
ZR = 128


def _leaky(x):
    return jnp.maximum(x, 0.01 * x)


def _make_msg_kernel(with_cnt):
    mesh = plsc.VectorSubcoreMesh(core_axis_name="c", subcore_axis_name="s")

    out_type = [jax.ShapeDtypeStruct((NC, NPAD, D), jnp.float32)]
    scratch = [
        pltpu.VMEM((EPW,), jnp.int32),
        pltpu.VMEM((EPW,), jnp.int32),
        pltpu.VMEM((CH * DE,), jnp.float32),
        pltpu.VMEM((CH * DE,), jnp.float32),
        pltpu.VMEM((CH, D), jnp.float32),
        pltpu.VMEM((CH, D), jnp.float32),
        pltpu.VMEM((DE, D), jnp.float32),
        pltpu.VMEM_SHARED((NPAD, D), jnp.float32),
        pltpu.SemaphoreType.DMA,
        pltpu.SemaphoreType.DMA,
        pltpu.SemaphoreType.DMA,
        pltpu.SemaphoreType.DMA,
    ]
    if with_cnt:
        out_type.append(jax.ShapeDtypeStruct((NC * NPAD,), jnp.float32))
        scratch += [
            pltpu.VMEM((CH,), jnp.float32),
            pltpu.VMEM((RPS,), jnp.float32),
            pltpu.VMEM_SHARED((NPAD,), jnp.float32),
        ]

    def body_common(table, edges, efr, wbot, out, cnt_out,
                    src_v, dst_v, ef_b, rows_b, w_v, agg, sg, ss,
                    ones_v, zc_v, cnt):
        c = lax.axis_index("c")
        s = lax.axis_index("s")
        wid = c * NS + s
        e0 = wid * EPW

        pltpu.sync_copy(edges.at[pl.ds(e0, EPW)], src_v)
        pltpu.sync_copy(edges.at[pl.ds(E + e0, EPW)], dst_v)
        pltpu.sync_copy(wbot, w_v)
        wvec = [[w_v[kk, pl.ds(chk * 16, 16)] for kk in range(DE)]
                for chk in range(D // 16)]

        zero = jnp.zeros((16,), jnp.float32)

        def zrow(r, carry):
            for chk in range(D // 16):
                rows_b[0][r, pl.ds(chk * 16, 16)] = zero
            return carry

        lax.fori_loop(0, CH, zrow, 0)
        row0 = s * RPS
        for b in range(RPS // CH):
            pltpu.sync_copy(rows_b[0], agg.at[pl.ds(row0 + b * CH, CH)])
        if with_cnt:
            one = jnp.full((16,), 1.0, jnp.float32)
            for g in range(CH // 16):
                ones_v[pl.ds(g * 16, 16)] = one

            def zc(r, carry):
                zc_v[pl.ds(r * 16, 16)] = zero
                return carry

            lax.fori_loop(0, RPS // 16, zc, 0)
            pltpu.sync_copy(zc_v, cnt.at[pl.ds(row0, RPS)])
        plsc.subcore_barrier()

        def issue(j, b):
            pltpu.async_copy(efr.at[pl.ds(e0 * DE + j * (CH * DE), CH * DE)],
                             ef_b[b], sg[b])
            pltpu.async_copy(table.at[src_v.at[pl.ds(j * CH, CH)]],
                             rows_b[b], sg[b])

        def wait_arrival(b):
            pltpu.make_async_copy(
                efr.at[pl.ds(0, CH * DE)], ef_b[b], sg[b]).wait()
            pltpu.make_async_copy(
                table.at[pl.ds(0, CH)], rows_b[b], sg[b]).wait()

        def issue_scatter(j, b):
            pltpu.async_copy(rows_b[b], agg.at[dst_v.at[pl.ds(j * CH, CH)]],
                             ss[b], add=True)
            if with_cnt:
                pltpu.async_copy(ones_v, cnt.at[dst_v.at[pl.ds(j * CH, CH)]],
                                 ss[b], add=True)

        def wait_scatter(b):
            pltpu.make_async_copy(
                rows_b[b], agg.at[pl.ds(0, CH)], ss[b]).wait()
            if with_cnt:
                pltpu.make_async_copy(
                    ones_v, cnt.at[pl.ds(0, CH)], ss[b]).wait()

        def compute(b):
            @plsc.parallel_loop(0, CH // 4, unroll=5)
            def group(gi):
                ev = ef_b[b][pl.ds(gi * 16, 16)]
                for ee in range(4):
                    r = gi * 4 + ee
                    e = [ev[ee * DE + kk] for kk in range(DE)]
                    for chk in range(D // 16):
                        sl = pl.ds(chk * 16, 16)
                        w = wvec[chk]
                        t0 = e[0] * w[0] + e[1] * w[1]
                        t1 = e[2] * w[2] + e[3] * w[3]
                        acc = rows_b[b][r, sl] + (t0 + t1)
                        rows_b[b][r, sl] = _leaky(acc)

        issue(0, 0)
        issue(1, 1)
        wait_arrival(0)
        compute(0)
        issue_scatter(0, 0)

        def pair(jj, carry):
            j1 = 2 * jj + 1
            wait_scatter(0)
            issue(j1 + 1, 0)
            wait_arrival(1)
            compute(1)
            issue_scatter(j1, 1)

            j2 = 2 * jj + 2
            wait_scatter(1)

            @pl.when(j2 + 1 < NCHUNK)
            def _():
                issue(j2 + 1, 1)

            wait_arrival(0)
            compute(0)
            issue_scatter(j2, 0)
            return carry

        lax.fori_loop(0, (NCHUNK - 1) // 2, pair, 0)
        wait_scatter(0)
        plsc.subcore_barrier()
        pltpu.sync_copy(agg.at[pl.ds(row0, RPS)], out.at[c, pl.ds(row0, RPS)])
        if with_cnt:
            pltpu.sync_copy(cnt.at[pl.ds(row0, RPS)],
                            cnt_out.at[pl.ds(c * NPAD + row0, RPS)])

    if with_cnt:
        def body_fn(table, edges, efr, wbot, out, cnt_out,
                    src_v, dst_v, ef0, ef1, rows0, rows1, w_v, agg,
                    sg0, sg1, ss0, ss1, ones_v, zc_v, cnt):
            body_common(table, edges, efr, wbot, out, cnt_out,
                        src_v, dst_v, (ef0, ef1), (rows0, rows1), w_v, agg,
                        (sg0, sg1), (ss0, ss1), ones_v, zc_v, cnt)
    else:
        def body_fn(table, edges, efr, wbot, out,
                    src_v, dst_v, ef0, ef1, rows0, rows1, w_v, agg,
                    sg0, sg1, ss0, ss1):
            body_common(table, edges, efr, wbot, out, None,
                        src_v, dst_v, (ef0, ef1), (rows0, rows1), w_v, agg,
                        (sg0, sg1), (ss0, ss1), None, None, None)

    return pl.kernel(
        body_fn,
        out_type=out_type if with_cnt else out_type[0],
        mesh=mesh,
        scratch_types=scratch,
    )


_msg_pass1 = _make_msg_kernel(True)
_msg_pass2 = _make_msg_kernel(False)


def _make_heads_kernel():
    mesh = plsc.VectorSubcoreMesh(core_axis_name="c", subcore_axis_name="s")

    @functools.partial(
        pl.kernel,
        out_type=[
            jax.ShapeDtypeStruct((E, D), jnp.float32),
            jax.ShapeDtypeStruct((NW, 8, 16), jnp.float32),
        ],
        mesh=mesh,
        scratch_types=[
            pltpu.VMEM((EPW,), jnp.int32),
            pltpu.VMEM((EPW,), jnp.int32),
            pltpu.VMEM((CH, D), jnp.float32),
            pltpu.VMEM((CH, D), jnp.float32),
            pltpu.VMEM((CH, D), jnp.float32),
            pltpu.VMEM((CH, D), jnp.float32),
            pltpu.VMEM((8, 16), jnp.float32),
            pltpu.SemaphoreType.DMA,
            pltpu.SemaphoreType.DMA,
            pltpu.SemaphoreType.DMA,
            pltpu.SemaphoreType.DMA,
        ],
    )
    def k(ts, td, edges, ps_out, stat_out,
          src_v, dst_v, rs0, rs1, rd0, rd1, stat_v, sg0, sg1, so0, so1):
        c = lax.axis_index("c")
        s = lax.axis_index("s")
        wid = c * NS + s
        e0 = wid * EPW
        rs_b = (rs0, rs1)
        rd_b = (rd0, rd1)
        sg = (sg0, sg1)
        so = (so0, so1)

        pltpu.sync_copy(edges.at[pl.ds(e0, EPW)], src_v)
        pltpu.sync_copy(edges.at[pl.ds(E + e0, EPW)], dst_v)

        zero = jnp.zeros((16,), jnp.float32)

        def issue(j, b):
            pltpu.async_copy(ts.at[src_v.at[pl.ds(j * CH, CH)]],
                             rs_b[b], sg[b])
            pltpu.async_copy(td.at[dst_v.at[pl.ds(j * CH, CH)]],
                             rd_b[b], sg[b])

        def wait_arrival(b):
            pltpu.make_async_copy(ts.at[pl.ds(0, CH)], rs_b[b], sg[b]).wait()
            pltpu.make_async_copy(td.at[pl.ds(0, CH)], rd_b[b], sg[b]).wait()

        def issue_write(j, b):
            pltpu.async_copy(rs_b[b],
                             ps_out.at[pl.ds(e0 + j * CH, CH)], so[b])

        def wait_write(b):
            pltpu.make_async_copy(rs_b[b], ps_out.at[pl.ds(0, CH)],
                                  so[b]).wait()

        def compute(b, cin):
            @plsc.parallel_loop(0, CH, unroll=4, carry=cin)
            def edge(r, cc):
                vals = []
                for chk in range(6):
                    sl = pl.ds(chk * 16, 16)
                    v = rs_b[b][r, sl] + rd_b[b][r, sl]
                    rs_b[b][r, sl] = v
                    if chk >= 2:
                        vals.append(v)
                return tuple(cc[i] + vals[i] for i in range(4)) + tuple(
                    cc[4 + i] + vals[i] * vals[i] for i in range(4))

            return edge

        issue(0, 0)
        issue(1, 1)
        wait_arrival(0)
        acc = compute(0, (zero,) * 8)
        issue_write(0, 0)

        def pair(jj, cin):
            j1 = 2 * jj + 1
            wait_write(0)
            issue(j1 + 1, 0)
            wait_arrival(1)
            cin = compute(1, cin)
            issue_write(j1, 1)

            j2 = 2 * jj + 2
            wait_write(1)

            @pl.when(j2 + 1 < NCHUNK)
            def _():
                issue(j2 + 1, 1)

            wait_arrival(0)
            cin = compute(0, cin)
            issue_write(j2, 0)
            return cin

        fin = lax.fori_loop(0, (NCHUNK - 1) // 2, pair, acc)
        wait_write(0)
        for i in range(8):
            stat_v[i, :] = fin[i]
        pltpu.sync_copy(stat_v, stat_out.at[wid])

    return k


_heads_pass = _make_heads_kernel()


_BN = 1000
_BE = 4000


def _table1(nfeats, wtop, bm):
    def body(nf, wt, b, out):
        out[...] = jnp.dot(nf[...], wt[...],
                           preferred_element_type=jnp.float32) + b[...]

    return pl.pallas_call(
        body,
        grid=(N // _BN,),
        in_specs=[
            pl.BlockSpec((_BN, D), lambda i: (i, 0)),
            pl.BlockSpec((D, D), lambda i: (0, 0)),
            pl.BlockSpec((1, D), lambda i: (0, 0)),
        ],
        out_specs=pl.BlockSpec((_BN, D), lambda i: (i, 0)),
        out_shape=jax.ShapeDtypeStruct((N, D), jnp.float32),
    )(nfeats, wtop, bm)


def _mid(p, cntb, nfeats, wa_t, wa_b, ba, wm2t, bm2):
    def body(p_b, cnt_b, nf, wat, wab, b, w2, b2, h1_o, q2_o):
        hn = (p_b[0] + p_b[1]) / cnt_b[...]
        h1 = _leaky(
            jnp.dot(nf[...], wat[...], preferred_element_type=jnp.float32)
            + jnp.dot(hn, wab[...], preferred_element_type=jnp.float32)
            + b[...])
        h1_o[...] = h1
        q2_o[...] = jnp.dot(h1, w2[...],
                            preferred_element_type=jnp.float32) + b2[...]

    return pl.pallas_call(
        body,
        grid=(N // _BN,),
        in_specs=[
            pl.BlockSpec((NC, _BN, D), lambda i: (0, i, 0)),
            pl.BlockSpec((_BN, D), lambda i: (i, 0)),
            pl.BlockSpec((_BN, D), lambda i: (i, 0)),
            pl.BlockSpec((D, D), lambda i: (0, 0)),
            pl.BlockSpec((D, D), lambda i: (0, 0)),
            pl.BlockSpec((1, D), lambda i: (0, 0)),
            pl.BlockSpec((D, D), lambda i: (0, 0)),
            pl.BlockSpec((1, D), lambda i: (0, 0)),
        ],
        out_specs=[
            pl.BlockSpec((_BN, D), lambda i: (i, 0)),
            pl.BlockSpec((_BN, D), lambda i: (i, 0)),
        ],
        out_shape=[
            jax.ShapeDtypeStruct((N, D), jnp.float32),
            jax.ShapeDtypeStruct((N, D), jnp.float32),
        ],
    )(p, cntb, nfeats, wa_t, wa_b, ba, wm2t, bm2)


def _tail_tables(p, cntb, h1, wa_t, wa_b, ba, wsrc, wdst, bcf):
    def body(p_b, cnt_b, h1_b, wat, wab, b, ws, wd, bb, ts_o, td_o):
        hn = (p_b[0] + p_b[1]) / cnt_b[...]
        h2 = _leaky(
            jnp.dot(h1_b[...], wat[...], preferred_element_type=jnp.float32)
            + jnp.dot(hn, wab[...], preferred_element_type=jnp.float32)
            + b[...])
        ts_o[...] = jnp.dot(h2, ws[...], preferred_element_type=jnp.float32)
        td_o[...] = jnp.dot(h2, wd[...],
                            preferred_element_type=jnp.float32) + bb[...]

    return pl.pallas_call(
        body,
        grid=(N // _BN,),
        in_specs=[
            pl.BlockSpec((NC, _BN, D), lambda i: (0, i, 0)),
            pl.BlockSpec((_BN, D), lambda i: (i, 0)),
            pl.BlockSpec((_BN, D), lambda i: (i, 0)),
            pl.BlockSpec((D, D), lambda i: (0, 0)),
            pl.BlockSpec((D, D), lambda i: (0, 0)),
            pl.BlockSpec((1, D), lambda i: (0, 0)),
            pl.BlockSpec((D, D), lambda i: (0, 0)),
            pl.BlockSpec((D, D), lambda i: (0, 0)),
            pl.BlockSpec((1, D), lambda i: (0, 0)),
        ],
        out_specs=[
            pl.BlockSpec((_BN, D), lambda i: (i, 0)),
            pl.BlockSpec((_BN, D), lambda i: (i, 0)),
        ],
        out_shape=[
            jax.ShapeDtypeStruct((N, D), jnp.float32),
            jax.ShapeDtypeStruct((N, D), jnp.float32),
        ],
    )(p, cntb, h1, wa_t, wa_b, ba, wsrc, wdst, bcf)


def _final_heads(ps, scale, shift, wc2, bc2, wf2, bf2):
    def body(x_b, sc, sh, wc, bc, wf, bf, co_o, fo_o):
        x = x_b[...]
        xc = jnp.maximum(x[:, :32], 0.0)
        co_o[...] = jnp.dot(xc, wc[...],
                            preferred_element_type=jnp.float32) + bc[...]
        yf = jnp.maximum(x[:, 32:96] * sc[...] + sh[...], 0.0)
        fo_o[...] = jnp.dot(yf, wf[...],
                            preferred_element_type=jnp.float32) + bf[...]

    return pl.pallas_call(
        body,
        grid=(E // _BE,),
        in_specs=[
            pl.BlockSpec((_BE, D), lambda i: (i, 0)),
            pl.BlockSpec((1, 64), lambda i: (0, 0)),
            pl.BlockSpec((1, 64), lambda i: (0, 0)),
            pl.BlockSpec((32, 2), lambda i: (0, 0)),
            pl.BlockSpec((1, 2), lambda i: (0, 0)),
            pl.BlockSpec((64, NFINE), lambda i: (0, 0)),
            pl.BlockSpec((1, NFINE), lambda i: (0, 0)),
        ],
        out_specs=[
            pl.BlockSpec((_BE, 2), lambda i: (i, 0)),
            pl.BlockSpec((_BE, NFINE), lambda i: (i, 0)),
        ],
        out_shape=[
            jax.ShapeDtypeStruct((E, 2), jnp.float32),
            jax.ShapeDtypeStruct((E, NFINE), jnp.float32),
        ],
    )(ps, scale, shift, wc2, bc2, wf2, bf2)


def kernel(nfeats, efeats, edge_index, W_msg1, b_msg1, W_app1, b_app1,
           W_msg2, b_msg2, W_app2, b_app2, Wc1, bc1, Wc2, bc2,
           Wf1, bf1, bn_g, bn_b, Wf2, bf2):
    edges = edge_index.reshape(2 * E)
    efr = efeats.reshape(E * DE)

    q1 = _table1(nfeats, W_msg1[:D], b_msg1.reshape(1, D))
    p1, cnt = _msg_pass1(q1, edges, efr, W_msg1[D:])
    cntb = jnp.broadcast_to(
        jnp.maximum(cnt[:N] + cnt[NPAD:NPAD + N], 1.0).reshape(N, 1), (N, D))
    h1, q2 = _mid(p1, cntb, nfeats,
                  W_app1[:D], W_app1[D:], b_app1.reshape(1, D),
                  W_msg2[:HID], b_msg2.reshape(1, D))

    p2 = _msg_pass2(q2, edges, efr, W_msg2[HID:])
    zpad = jnp.zeros((D, 32), jnp.float32)
    wsrc = jnp.concatenate([Wc1[:D], Wf1[:D], zpad], axis=1)
    wdst = jnp.concatenate([Wc1[D:], Wf1[D:], zpad], axis=1)
    bcf = jnp.concatenate([bc1, bf1, jnp.zeros((32,), jnp.float32)])
    tsrc, tdst = _tail_tables(p2, cntb, h1,
                              W_app2[:HID], W_app2[HID:],
                              b_app2.reshape(1, D), wsrc, wdst,
                              bcf.reshape(1, D))

    ps, stats = _heads_pass(tsrc, tdst, edges)
    ssum = stats[:, 0:4, :].sum(axis=0).reshape(64)
    ssq = stats[:, 4:8, :].sum(axis=0).reshape(64)
    mean = ssum / E
    var = ssq / E - mean * mean
    scale = bn_g * lax.rsqrt(var + 1e-5)
    shift = bn_b - mean * scale
    coarse, fine = _final_heads(ps, scale.reshape(1, 64), shift.reshape(1, 64),
                                Wc2, bc2.reshape(1, 2), Wf2,
                                bf2.reshape(1, NFINE))
    return coarse, fine


# speedup vs baseline: 1.0029x; 1.0029x over previous
import functools

import jax
import jax.numpy as jnp
from jax import lax
from jax.experimental import pallas as pl
from jax.experimental.pallas import tpu as pltpu
from jax.experimental.pallas import tpu_sc as plsc

N = 10000
E = 320000
D = 128
DE = 4
HID = 128
NFINE = 10

NC = 2
NS = 16
NW = NC * NS
EPW = E // NW
CH = 80
NCHUNK = EPW // CH
NPAD = 10240
RPS = NPAD // NS
ZR = 128


def _leaky(x):
    return jnp.maximum(x, 0.01 * x)


def _make_msg_kernel(with_cnt):
    mesh = plsc.VectorSubcoreMesh(core_axis_name="c", subcore_axis_name="s")

    out_type = [jax.ShapeDtypeStruct((NC, NPAD, D), jnp.float32)]
    scratch = [
        pltpu.VMEM((EPW,), jnp.int32),
        pltpu.VMEM((EPW,), jnp.int32),
        pltpu.VMEM((CH * DE,), jnp.float32),
        pltpu.VMEM((CH * DE,), jnp.float32),
        pltpu.VMEM((CH, D), jnp.float32),
        pltpu.VMEM((CH, D), jnp.float32),
        pltpu.VMEM((DE, D), jnp.float32),
        pltpu.VMEM_SHARED((NPAD, D), jnp.float32),
        pltpu.SemaphoreType.DMA,
        pltpu.SemaphoreType.DMA,
        pltpu.SemaphoreType.DMA,
        pltpu.SemaphoreType.DMA,
    ]
    if with_cnt:
        out_type.append(jax.ShapeDtypeStruct((NC * NPAD,), jnp.float32))
        scratch += [
            pltpu.VMEM((CH,), jnp.float32),
            pltpu.VMEM((RPS,), jnp.float32),
            pltpu.VMEM_SHARED((NPAD,), jnp.float32),
        ]

    def body_common(table, edges, efr, wbot, out, cnt_out,
                    src_v, dst_v, ef_b, rows_b, w_v, agg, sg, ss,
                    ones_v, zc_v, cnt):
        c = lax.axis_index("c")
        s = lax.axis_index("s")
        wid = c * NS + s
        e0 = wid * EPW

        pltpu.sync_copy(edges.at[pl.ds(e0, EPW)], src_v)
        pltpu.sync_copy(edges.at[pl.ds(E + e0, EPW)], dst_v)
        pltpu.sync_copy(wbot, w_v)
        wvec = [[w_v[kk, pl.ds(chk * 16, 16)] for kk in range(DE)]
                for chk in range(D // 16)]

        zero = jnp.zeros((16,), jnp.float32)

        def zrow(r, carry):
            for chk in range(D // 16):
                rows_b[0][r, pl.ds(chk * 16, 16)] = zero
            return carry

        lax.fori_loop(0, CH, zrow, 0)
        row0 = s * RPS
        for b in range(RPS // CH):
            pltpu.sync_copy(rows_b[0], agg.at[pl.ds(row0 + b * CH, CH)])
        if with_cnt:
            one = jnp.full((16,), 1.0, jnp.float32)
            for g in range(CH // 16):
                ones_v[pl.ds(g * 16, 16)] = one

            def zc(r, carry):
                zc_v[pl.ds(r * 16, 16)] = zero
                return carry

            lax.fori_loop(0, RPS // 16, zc, 0)
            pltpu.sync_copy(zc_v, cnt.at[pl.ds(row0, RPS)])
        plsc.subcore_barrier()

        def issue(j, b):
            pltpu.async_copy(efr.at[pl.ds(e0 * DE + j * (CH * DE), CH * DE)],
                             ef_b[b], sg[b])
            pltpu.async_copy(table.at[src_v.at[pl.ds(j * CH, CH)]],
                             rows_b[b], sg[b])

        def wait_arrival(b):
            pltpu.make_async_copy(
                efr.at[pl.ds(0, CH * DE)], ef_b[b], sg[b]).wait()
            pltpu.make_async_copy(
                table.at[pl.ds(0, CH)], rows_b[b], sg[b]).wait()

        def issue_scatter(j, b):
            pltpu.async_copy(rows_b[b], agg.at[dst_v.at[pl.ds(j * CH, CH)]],
                             ss[b], add=True)
            if with_cnt:
                pltpu.async_copy(ones_v, cnt.at[dst_v.at[pl.ds(j * CH, CH)]],
                                 ss[b], add=True)

        def wait_scatter(b):
            pltpu.make_async_copy(
                rows_b[b], agg.at[pl.ds(0, CH)], ss[b]).wait()
            if with_cnt:
                pltpu.make_async_copy(
                    ones_v, cnt.at[pl.ds(0, CH)], ss[b]).wait()

        def compute(b):
            @plsc.parallel_loop(0, CH // 4, unroll=4)
            def group(gi):
                ev = ef_b[b][pl.ds(gi * 16, 16)]
                for ee in range(4):
                    r = gi * 4 + ee
                    e = [ev[ee * DE + kk] for kk in range(DE)]
                    for chk in range(D // 16):
                        sl = pl.ds(chk * 16, 16)
                        w = wvec[chk]
                        t0 = e[0] * w[0] + e[1] * w[1]
                        t1 = e[2] * w[2] + e[3] * w[3]
                        acc = rows_b[b][r, sl] + (t0 + t1)
                        rows_b[b][r, sl] = _leaky(acc)

        issue(0, 0)
        issue(1, 1)
        wait_arrival(0)
        compute(0)
        issue_scatter(0, 0)

        def pair(jj, carry):
            j1 = 2 * jj + 1
            wait_scatter(0)
            issue(j1 + 1, 0)
            wait_arrival(1)
            compute(1)
            issue_scatter(j1, 1)

            j2 = 2 * jj + 2
            wait_scatter(1)

            @pl.when(j2 + 1 < NCHUNK)
            def _():
                issue(j2 + 1, 1)

            wait_arrival(0)
            compute(0)
            issue_scatter(j2, 0)
            return carry

        lax.fori_loop(0, (NCHUNK - 1) // 2, pair, 0)
        wait_scatter(0)
        plsc.subcore_barrier()
        pltpu.sync_copy(agg.at[pl.ds(row0, RPS)], out.at[c, pl.ds(row0, RPS)])
        if with_cnt:
            pltpu.sync_copy(cnt.at[pl.ds(row0, RPS)],
                            cnt_out.at[pl.ds(c * NPAD + row0, RPS)])

    if with_cnt:
        def body_fn(table, edges, efr, wbot, out, cnt_out,
                    src_v, dst_v, ef0, ef1, rows0, rows1, w_v, agg,
                    sg0, sg1, ss0, ss1, ones_v, zc_v, cnt):
            body_common(table, edges, efr, wbot, out, cnt_out,
                        src_v, dst_v, (ef0, ef1), (rows0, rows1), w_v, agg,
                        (sg0, sg1), (ss0, ss1), ones_v, zc_v, cnt)
    else:
        def body_fn(table, edges, efr, wbot, out,
                    src_v, dst_v, ef0, ef1, rows0, rows1, w_v, agg,
                    sg0, sg1, ss0, ss1):
            body_common(table, edges, efr, wbot, out, None,
                        src_v, dst_v, (ef0, ef1), (rows0, rows1), w_v, agg,
                        (sg0, sg1), (ss0, ss1), None, None, None)

    return pl.kernel(
        body_fn,
        out_type=out_type if with_cnt else out_type[0],
        mesh=mesh,
        scratch_types=scratch,
    )


_msg_pass1 = _make_msg_kernel(True)
_msg_pass2 = _make_msg_kernel(False)


def _make_heads_kernel():
    mesh = plsc.VectorSubcoreMesh(core_axis_name="c", subcore_axis_name="s")

    @functools.partial(
        pl.kernel,
        out_type=[
            jax.ShapeDtypeStruct((E, D), jnp.float32),
            jax.ShapeDtypeStruct((NW, 8, 16), jnp.float32),
        ],
        mesh=mesh,
        scratch_types=[
            pltpu.VMEM((EPW,), jnp.int32),
            pltpu.VMEM((EPW,), jnp.int32),
            pltpu.VMEM((CH, D), jnp.float32),
            pltpu.VMEM((CH, D), jnp.float32),
            pltpu.VMEM((CH, D), jnp.float32),
            pltpu.VMEM((CH, D), jnp.float32),
            pltpu.VMEM((8, 16), jnp.float32),
            pltpu.SemaphoreType.DMA,
            pltpu.SemaphoreType.DMA,
            pltpu.SemaphoreType.DMA,
            pltpu.SemaphoreType.DMA,
        ],
    )
    def k(ts, td, edges, ps_out, stat_out,
          src_v, dst_v, rs0, rs1, rd0, rd1, stat_v, sg0, sg1, so0, so1):
        c = lax.axis_index("c")
        s = lax.axis_index("s")
        wid = c * NS + s
        e0 = wid * EPW
        rs_b = (rs0, rs1)
        rd_b = (rd0, rd1)
        sg = (sg0, sg1)
        so = (so0, so1)

        pltpu.sync_copy(edges.at[pl.ds(e0, EPW)], src_v)
        pltpu.sync_copy(edges.at[pl.ds(E + e0, EPW)], dst_v)

        zero = jnp.zeros((16,), jnp.float32)

        def issue(j, b):
            pltpu.async_copy(ts.at[src_v.at[pl.ds(j * CH, CH)]],
                             rs_b[b], sg[b])
            pltpu.async_copy(td.at[dst_v.at[pl.ds(j * CH, CH)]],
                             rd_b[b], sg[b])

        def wait_arrival(b):
            pltpu.make_async_copy(ts.at[pl.ds(0, CH)], rs_b[b], sg[b]).wait()
            pltpu.make_async_copy(td.at[pl.ds(0, CH)], rd_b[b], sg[b]).wait()

        def issue_write(j, b):
            pltpu.async_copy(rs_b[b],
                             ps_out.at[pl.ds(e0 + j * CH, CH)], so[b])

        def wait_write(b):
            pltpu.make_async_copy(rs_b[b], ps_out.at[pl.ds(0, CH)],
                                  so[b]).wait()

        def compute(b, cin):
            @plsc.parallel_loop(0, CH, unroll=4, carry=cin)
            def edge(r, cc):
                vals = []
                for chk in range(6):
                    sl = pl.ds(chk * 16, 16)
                    v = rs_b[b][r, sl] + rd_b[b][r, sl]
                    rs_b[b][r, sl] = v
                    if chk >= 2:
                        vals.append(v)
                return tuple(cc[i] + vals[i] for i in range(4)) + tuple(
                    cc[4 + i] + vals[i] * vals[i] for i in range(4))

            return edge

        issue(0, 0)
        issue(1, 1)
        wait_arrival(0)
        acc = compute(0, (zero,) * 8)
        issue_write(0, 0)

        def pair(jj, cin):
            j1 = 2 * jj + 1
            wait_write(0)
            issue(j1 + 1, 0)
            wait_arrival(1)
            cin = compute(1, cin)
            issue_write(j1, 1)

            j2 = 2 * jj + 2
            wait_write(1)

            @pl.when(j2 + 1 < NCHUNK)
            def _():
                issue(j2 + 1, 1)

            wait_arrival(0)
            cin = compute(0, cin)
            issue_write(j2, 0)
            return cin

        fin = lax.fori_loop(0, (NCHUNK - 1) // 2, pair, acc)
        wait_write(0)
        for i in range(8):
            stat_v[i, :] = fin[i]
        pltpu.sync_copy(stat_v, stat_out.at[wid])

    return k


_heads_pass = _make_heads_kernel()


_BN = 1000
_BE = 4000


def _table1(nfeats, wtop, bm):
    def body(nf, wt, b, out):
        out[...] = jnp.dot(nf[...], wt[...],
                           preferred_element_type=jnp.float32) + b[...]

    return pl.pallas_call(
        body,
        grid=(N // _BN,),
        in_specs=[
            pl.BlockSpec((_BN, D), lambda i: (i, 0)),
            pl.BlockSpec((D, D), lambda i: (0, 0)),
            pl.BlockSpec((1, D), lambda i: (0, 0)),
        ],
        out_specs=pl.BlockSpec((_BN, D), lambda i: (i, 0)),
        out_shape=jax.ShapeDtypeStruct((N, D), jnp.float32),
    )(nfeats, wtop, bm)


def _mid(p, cntb, nfeats, wa_t, wa_b, ba, wm2t, bm2):
    def body(p_b, cnt_b, nf, wat, wab, b, w2, b2, h1_o, q2_o):
        hn = (p_b[0] + p_b[1]) / cnt_b[...]
        h1 = _leaky(
            jnp.dot(nf[...], wat[...], preferred_element_type=jnp.float32)
            + jnp.dot(hn, wab[...], preferred_element_type=jnp.float32)
            + b[...])
        h1_o[...] = h1
        q2_o[...] = jnp.dot(h1, w2[...],
                            preferred_element_type=jnp.float32) + b2[...]

    return pl.pallas_call(
        body,
        grid=(N // _BN,),
        in_specs=[
            pl.BlockSpec((NC, _BN, D), lambda i: (0, i, 0)),
            pl.BlockSpec((_BN, D), lambda i: (i, 0)),
            pl.BlockSpec((_BN, D), lambda i: (i, 0)),
            pl.BlockSpec((D, D), lambda i: (0, 0)),
            pl.BlockSpec((D, D), lambda i: (0, 0)),
            pl.BlockSpec((1, D), lambda i: (0, 0)),
            pl.BlockSpec((D, D), lambda i: (0, 0)),
            pl.BlockSpec((1, D), lambda i: (0, 0)),
        ],
        out_specs=[
            pl.BlockSpec((_BN, D), lambda i: (i, 0)),
            pl.BlockSpec((_BN, D), lambda i: (i, 0)),
        ],
        out_shape=[
            jax.ShapeDtypeStruct((N, D), jnp.float32),
            jax.ShapeDtypeStruct((N, D), jnp.float32),
        ],
    )(p, cntb, nfeats, wa_t, wa_b, ba, wm2t, bm2)


def _tail_tables(p, cntb, h1, wa_t, wa_b, ba, wsrc, wdst, bcf):
    def body(p_b, cnt_b, h1_b, wat, wab, b, ws, wd, bb, ts_o, td_o):
        hn = (p_b[0] + p_b[1]) / cnt_b[...]
        h2 = _leaky(
            jnp.dot(h1_b[...], wat[...], preferred_element_type=jnp.float32)
            + jnp.dot(hn, wab[...], preferred_element_type=jnp.float32)
            + b[...])
        ts_o[...] = jnp.dot(h2, ws[...], preferred_element_type=jnp.float32)
        td_o[...] = jnp.dot(h2, wd[...],
                            preferred_element_type=jnp.float32) + bb[...]

    return pl.pallas_call(
        body,
        grid=(N // _BN,),
        in_specs=[
            pl.BlockSpec((NC, _BN, D), lambda i: (0, i, 0)),
            pl.BlockSpec((_BN, D), lambda i: (i, 0)),
            pl.BlockSpec((_BN, D), lambda i: (i, 0)),
            pl.BlockSpec((D, D), lambda i: (0, 0)),
            pl.BlockSpec((D, D), lambda i: (0, 0)),
            pl.BlockSpec((1, D), lambda i: (0, 0)),
            pl.BlockSpec((D, D), lambda i: (0, 0)),
            pl.BlockSpec((D, D), lambda i: (0, 0)),
            pl.BlockSpec((1, D), lambda i: (0, 0)),
        ],
        out_specs=[
            pl.BlockSpec((_BN, D), lambda i: (i, 0)),
            pl.BlockSpec((_BN, D), lambda i: (i, 0)),
        ],
        out_shape=[
            jax.ShapeDtypeStruct((N, D), jnp.float32),
            jax.ShapeDtypeStruct((N, D), jnp.float32),
        ],
    )(p, cntb, h1, wa_t, wa_b, ba, wsrc, wdst, bcf)


def _final_heads(ps, scale, shift, wc2, bc2, wf2, bf2):
    def body(x_b, sc, sh, wc, bc, wf, bf, co_o, fo_o):
        x = x_b[...]
        xc = jnp.maximum(x[:, :32], 0.0)
        co_o[...] = jnp.dot(xc, wc[...],
                            preferred_element_type=jnp.float32) + bc[...]
        yf = jnp.maximum(x[:, 32:96] * sc[...] + sh[...], 0.0)
        fo_o[...] = jnp.dot(yf, wf[...],
                            preferred_element_type=jnp.float32) + bf[...]

    return pl.pallas_call(
        body,
        grid=(E // _BE,),
        in_specs=[
            pl.BlockSpec((_BE, D), lambda i: (i, 0)),
            pl.BlockSpec((1, 64), lambda i: (0, 0)),
            pl.BlockSpec((1, 64), lambda i: (0, 0)),
            pl.BlockSpec((32, 2), lambda i: (0, 0)),
            pl.BlockSpec((1, 2), lambda i: (0, 0)),
            pl.BlockSpec((64, NFINE), lambda i: (0, 0)),
            pl.BlockSpec((1, NFINE), lambda i: (0, 0)),
        ],
        out_specs=[
            pl.BlockSpec((_BE, 2), lambda i: (i, 0)),
            pl.BlockSpec((_BE, NFINE), lambda i: (i, 0)),
        ],
        out_shape=[
            jax.ShapeDtypeStruct((E, 2), jnp.float32),
            jax.ShapeDtypeStruct((E, NFINE), jnp.float32),
        ],
    )(ps, scale, shift, wc2, bc2, wf2, bf2)


def kernel(nfeats, efeats, edge_index, W_msg1, b_msg1, W_app1, b_app1,
           W_msg2, b_msg2, W_app2, b_app2, Wc1, bc1, Wc2, bc2,
           Wf1, bf1, bn_g, bn_b, Wf2, bf2):
    edges = edge_index.reshape(2 * E)
    efr = efeats.reshape(E * DE)

    q1 = _table1(nfeats, W_msg1[:D], b_msg1.reshape(1, D))
    p1, cnt = _msg_pass1(q1, edges, efr, W_msg1[D:])
    cntb = jnp.broadcast_to(
        jnp.maximum(cnt[:N] + cnt[NPAD:NPAD + N], 1.0).reshape(N, 1), (N, D))
    h1, q2 = _mid(p1, cntb, nfeats,
                  W_app1[:D], W_app1[D:], b_app1.reshape(1, D),
                  W_msg2[:HID], b_msg2.reshape(1, D))

    p2 = _msg_pass2(q2, edges, efr, W_msg2[HID:])
    zpad = jnp.zeros((D, 32), jnp.float32)
    wsrc = jnp.concatenate([Wc1[:D], Wf1[:D], zpad], axis=1)
    wdst = jnp.concatenate([Wc1[D:], Wf1[D:], zpad], axis=1)
    bcf = jnp.concatenate([bc1, bf1, jnp.zeros((32,), jnp.float32)])
    tsrc, tdst = _tail_tables(p2, cntb, h1,
                              W_app2[:HID], W_app2[HID:],
                              b_app2.reshape(1, D), wsrc, wdst,
                              bcf.reshape(1, D))

    ps, stats = _heads_pass(tsrc, tdst, edges)
    ssum = stats[:, 0:4, :].sum(axis=0).reshape(64)
    ssq = stats[:, 4:8, :].sum(axis=0).reshape(64)
    mean = ssum / E
    var = ssq / E - mean * mean
    scale = bn_g * lax.rsqrt(var + 1e-5)
    shift = bn_b - mean * scale
    coarse, fine = _final_heads(ps, scale.reshape(1, 64), shift.reshape(1, 64),
                                Wc2, bc2.reshape(1, 2), Wf2,
                                bf2.reshape(1, NFINE))
    return coarse, fine


# speedup vs baseline: 1.0129x; 1.0099x over previous
import functools

import jax
import jax.numpy as jnp
from jax import lax
from jax.experimental import pallas as pl
from jax.experimental.pallas import tpu as pltpu
from jax.experimental.pallas import tpu_sc as plsc

N = 10000
E = 320000
D = 128
DE = 4
HID = 128
NFINE = 10

NC = 2
NS = 16
NW = NC * NS
EPW = E // NW
CH = 80
NCHUNK = EPW // CH
NPAD = 10240
RPS = NPAD // NS
ZR = 128


def _leaky(x):
    return jnp.maximum(x, 0.01 * x)


def _make_msg_kernel(with_cnt):
    mesh = plsc.VectorSubcoreMesh(core_axis_name="c", subcore_axis_name="s")

    out_type = [jax.ShapeDtypeStruct((NC, NPAD, D), jnp.float32)]
    scratch = [
        pltpu.VMEM((EPW,), jnp.int32),
        pltpu.VMEM((EPW,), jnp.int32),
        pltpu.VMEM((CH * DE,), jnp.float32),
        pltpu.VMEM((CH * DE,), jnp.float32),
        pltpu.VMEM((CH, D), jnp.float32),
        pltpu.VMEM((CH, D), jnp.float32),
        pltpu.VMEM((DE, D), jnp.float32),
        pltpu.VMEM_SHARED((NPAD, D), jnp.float32),
        pltpu.SemaphoreType.DMA,
        pltpu.SemaphoreType.DMA,
        pltpu.SemaphoreType.DMA,
        pltpu.SemaphoreType.DMA,
    ]
    if with_cnt:
        out_type.append(jax.ShapeDtypeStruct((NC * NPAD,), jnp.float32))
        scratch += [
            pltpu.VMEM((CH,), jnp.float32),
            pltpu.VMEM((RPS,), jnp.float32),
            pltpu.VMEM_SHARED((NPAD,), jnp.float32),
        ]

    def body_common(table, edges, efr, wbot, out, cnt_out,
                    src_v, dst_v, ef_b, rows_b, w_v, agg, sg, ss,
                    ones_v, zc_v, cnt):
        c = lax.axis_index("c")
        s = lax.axis_index("s")
        wid = c * NS + s
        e0 = wid * EPW

        pltpu.sync_copy(edges.at[pl.ds(e0, EPW)], src_v)
        pltpu.sync_copy(edges.at[pl.ds(E + e0, EPW)], dst_v)
        pltpu.sync_copy(wbot, w_v)
        wvec = [[w_v[kk, pl.ds(chk * 16, 16)] for kk in range(DE)]
                for chk in range(D // 16)]

        zero = jnp.zeros((16,), jnp.float32)

        def zrow(r, carry):
            for chk in range(D // 16):
                rows_b[0][r, pl.ds(chk * 16, 16)] = zero
            return carry

        lax.fori_loop(0, CH, zrow, 0)
        row0 = s * RPS
        for b in range(RPS // CH):
            pltpu.sync_copy(rows_b[0], agg.at[pl.ds(row0 + b * CH, CH)])
        if with_cnt:
            one = jnp.full((16,), 1.0, jnp.float32)
            for g in range(CH // 16):
                ones_v[pl.ds(g * 16, 16)] = one

            def zc(r, carry):
                zc_v[pl.ds(r * 16, 16)] = zero
                return carry

            lax.fori_loop(0, RPS // 16, zc, 0)
            pltpu.sync_copy(zc_v, cnt.at[pl.ds(row0, RPS)])
        plsc.subcore_barrier()

        def issue(j, b):
            pltpu.async_copy(efr.at[pl.ds(e0 * DE + j * (CH * DE), CH * DE)],
                             ef_b[b], sg[b])
            pltpu.async_copy(table.at[src_v.at[pl.ds(j * CH, CH)]],
                             rows_b[b], sg[b])

        def wait_arrival(b):
            pltpu.make_async_copy(
                efr.at[pl.ds(0, CH * DE)], ef_b[b], sg[b]).wait()
            pltpu.make_async_copy(
                table.at[pl.ds(0, CH)], rows_b[b], sg[b]).wait()

        def issue_scatter(j, b):
            pltpu.async_copy(rows_b[b], agg.at[dst_v.at[pl.ds(j * CH, CH)]],
                             ss[b], add=True)
            if with_cnt:
                pltpu.async_copy(ones_v, cnt.at[dst_v.at[pl.ds(j * CH, CH)]],
                                 ss[b], add=True)

        def wait_scatter(b):
            pltpu.make_async_copy(
                rows_b[b], agg.at[pl.ds(0, CH)], ss[b]).wait()
            if with_cnt:
                pltpu.make_async_copy(
                    ones_v, cnt.at[pl.ds(0, CH)], ss[b]).wait()

        def compute(b):
            @plsc.parallel_loop(0, CH // 4, unroll=4)
            def group(gi):
                ev = ef_b[b][pl.ds(gi * 16, 16)]
                for ee in range(4):
                    r = gi * 4 + ee
                    e = [ev[ee * DE + kk] for kk in range(DE)]
                    for chk in range(D // 16):
                        sl = pl.ds(chk * 16, 16)
                        w = wvec[chk]
                        t0 = e[0] * w[0] + e[1] * w[1]
                        t1 = e[2] * w[2] + e[3] * w[3]
                        acc = rows_b[b][r, sl] + (t0 + t1)
                        rows_b[b][r, sl] = _leaky(acc)

        issue(0, 0)
        issue(1, 1)
        wait_arrival(0)
        compute(0)
        issue_scatter(0, 0)

        def pair(jj, carry):
            j1 = 2 * jj + 1
            wait_scatter(0)
            issue(j1 + 1, 0)
            wait_arrival(1)
            compute(1)
            issue_scatter(j1, 1)

            j2 = 2 * jj + 2
            wait_scatter(1)

            @pl.when(j2 + 1 < NCHUNK)
            def _():
                issue(j2 + 1, 1)

            wait_arrival(0)
            compute(0)
            issue_scatter(j2, 0)
            return carry

        lax.fori_loop(0, (NCHUNK - 1) // 2, pair, 0)
        wait_scatter(0)
        plsc.subcore_barrier()
        pltpu.sync_copy(agg.at[pl.ds(row0, RPS)], out.at[c, pl.ds(row0, RPS)])
        if with_cnt:
            pltpu.sync_copy(cnt.at[pl.ds(row0, RPS)],
                            cnt_out.at[pl.ds(c * NPAD + row0, RPS)])

    if with_cnt:
        def body_fn(table, edges, efr, wbot, out, cnt_out,
                    src_v, dst_v, ef0, ef1, rows0, rows1, w_v, agg,
                    sg0, sg1, ss0, ss1, ones_v, zc_v, cnt):
            body_common(table, edges, efr, wbot, out, cnt_out,
                        src_v, dst_v, (ef0, ef1), (rows0, rows1), w_v, agg,
                        (sg0, sg1), (ss0, ss1), ones_v, zc_v, cnt)
    else:
        def body_fn(table, edges, efr, wbot, out,
                    src_v, dst_v, ef0, ef1, rows0, rows1, w_v, agg,
                    sg0, sg1, ss0, ss1):
            body_common(table, edges, efr, wbot, out, None,
                        src_v, dst_v, (ef0, ef1), (rows0, rows1), w_v, agg,
                        (sg0, sg1), (ss0, ss1), None, None, None)

    return pl.kernel(
        body_fn,
        out_type=out_type if with_cnt else out_type[0],
        mesh=mesh,
        scratch_types=scratch,
    )


_msg_pass1 = _make_msg_kernel(True)
_msg_pass2 = _make_msg_kernel(False)


CHH = 200
NCHH = EPW // CHH


def _make_heads_kernel():
    mesh = plsc.VectorSubcoreMesh(core_axis_name="c", subcore_axis_name="s")

    @functools.partial(
        pl.kernel,
        out_type=[
            jax.ShapeDtypeStruct((E, D), jnp.float32),
            jax.ShapeDtypeStruct((NW, 8, 16), jnp.float32),
        ],
        mesh=mesh,
        scratch_types=[
            pltpu.VMEM((EPW,), jnp.int32),
            pltpu.VMEM((EPW,), jnp.int32),
            pltpu.VMEM((CHH, D), jnp.float32),
            pltpu.VMEM((CHH, D), jnp.float32),
            pltpu.VMEM((CHH, D), jnp.float32),
            pltpu.VMEM((CHH, D), jnp.float32),
            pltpu.VMEM((8, 16), jnp.float32),
            pltpu.SemaphoreType.DMA,
            pltpu.SemaphoreType.DMA,
            pltpu.SemaphoreType.DMA,
            pltpu.SemaphoreType.DMA,
        ],
    )
    def k(ts, td, edges, ps_out, stat_out,
          src_v, dst_v, rs0, rs1, rd0, rd1, stat_v, sg0, sg1, so0, so1):
        c = lax.axis_index("c")
        s = lax.axis_index("s")
        wid = c * NS + s
        e0 = wid * EPW
        rs_b = (rs0, rs1)
        rd_b = (rd0, rd1)
        sg = (sg0, sg1)
        so = (so0, so1)

        pltpu.sync_copy(edges.at[pl.ds(e0, EPW)], src_v)
        pltpu.sync_copy(edges.at[pl.ds(E + e0, EPW)], dst_v)

        zero = jnp.zeros((16,), jnp.float32)

        def issue(j, b):
            pltpu.async_copy(ts.at[src_v.at[pl.ds(j * CHH, CHH)]],
                             rs_b[b], sg[b])
            pltpu.async_copy(td.at[dst_v.at[pl.ds(j * CHH, CHH)]],
                             rd_b[b], sg[b])

        def wait_arrival(b):
            pltpu.make_async_copy(ts.at[pl.ds(0, CHH)], rs_b[b], sg[b]).wait()
            pltpu.make_async_copy(td.at[pl.ds(0, CHH)], rd_b[b], sg[b]).wait()

        def issue_write(j, b):
            pltpu.async_copy(rs_b[b],
                             ps_out.at[pl.ds(e0 + j * CHH, CHH)], so[b])

        def wait_write(b):
            pltpu.make_async_copy(rs_b[b], ps_out.at[pl.ds(0, CHH)],
                                  so[b]).wait()

        def compute(b, cin):
            @plsc.parallel_loop(0, CHH, unroll=4, carry=cin)
            def edge(r, cc):
                vals = []
                for chk in range(6):
                    sl = pl.ds(chk * 16, 16)
                    v = rs_b[b][r, sl] + rd_b[b][r, sl]
                    rs_b[b][r, sl] = v
                    if chk >= 2:
                        vals.append(v)
                return tuple(cc[i] + vals[i] for i in range(4)) + tuple(
                    cc[4 + i] + vals[i] * vals[i] for i in range(4))

            return edge

        issue(0, 0)
        issue(1, 1)
        wait_arrival(0)
        acc = compute(0, (zero,) * 8)
        issue_write(0, 0)

        def pair(jj, cin):
            j1 = 2 * jj + 1
            wait_write(0)
            issue(j1 + 1, 0)
            wait_arrival(1)
            cin = compute(1, cin)
            issue_write(j1, 1)

            j2 = 2 * jj + 2
            wait_write(1)

            @pl.when(j2 + 1 < NCHH)
            def _():
                issue(j2 + 1, 1)

            wait_arrival(0)
            cin = compute(0, cin)
            issue_write(j2, 0)
            return cin

        fin = lax.fori_loop(0, (NCHH - 2) // 2, pair, acc)
        wait_arrival(1)
        fin = compute(1, fin)
        issue_write(NCHH - 1, 1)
        wait_write(0)
        wait_write(1)
        for i in range(8):
            stat_v[i, :] = fin[i]
        pltpu.sync_copy(stat_v, stat_out.at[wid])

    return k


_heads_pass = _make_heads_kernel()


_BN = 1000
_BE = 4000


def _table1(nfeats, wtop, bm):
    def body(nf, wt, b, out):
        out[...] = jnp.dot(nf[...], wt[...],
                           preferred_element_type=jnp.float32) + b[...]

    return pl.pallas_call(
        body,
        grid=(N // _BN,),
        in_specs=[
            pl.BlockSpec((_BN, D), lambda i: (i, 0)),
            pl.BlockSpec((D, D), lambda i: (0, 0)),
            pl.BlockSpec((1, D), lambda i: (0, 0)),
        ],
        out_specs=pl.BlockSpec((_BN, D), lambda i: (i, 0)),
        out_shape=jax.ShapeDtypeStruct((N, D), jnp.float32),
    )(nfeats, wtop, bm)


def _mid(p, cntb, nfeats, wa_t, wa_b, ba, wm2t, bm2):
    def body(p_b, cnt_b, nf, wat, wab, b, w2, b2, h1_o, q2_o):
        hn = (p_b[0] + p_b[1]) / cnt_b[...]
        h1 = _leaky(
            jnp.dot(nf[...], wat[...], preferred_element_type=jnp.float32)
            + jnp.dot(hn, wab[...], preferred_element_type=jnp.float32)
            + b[...])
        h1_o[...] = h1
        q2_o[...] = jnp.dot(h1, w2[...],
                            preferred_element_type=jnp.float32) + b2[...]

    return pl.pallas_call(
        body,
        grid=(N // _BN,),
        in_specs=[
            pl.BlockSpec((NC, _BN, D), lambda i: (0, i, 0)),
            pl.BlockSpec((_BN, D), lambda i: (i, 0)),
            pl.BlockSpec((_BN, D), lambda i: (i, 0)),
            pl.BlockSpec((D, D), lambda i: (0, 0)),
            pl.BlockSpec((D, D), lambda i: (0, 0)),
            pl.BlockSpec((1, D), lambda i: (0, 0)),
            pl.BlockSpec((D, D), lambda i: (0, 0)),
            pl.BlockSpec((1, D), lambda i: (0, 0)),
        ],
        out_specs=[
            pl.BlockSpec((_BN, D), lambda i: (i, 0)),
            pl.BlockSpec((_BN, D), lambda i: (i, 0)),
        ],
        out_shape=[
            jax.ShapeDtypeStruct((N, D), jnp.float32),
            jax.ShapeDtypeStruct((N, D), jnp.float32),
        ],
    )(p, cntb, nfeats, wa_t, wa_b, ba, wm2t, bm2)


def _tail_tables(p, cntb, h1, wa_t, wa_b, ba, wsrc, wdst, bcf):
    def body(p_b, cnt_b, h1_b, wat, wab, b, ws, wd, bb, ts_o, td_o):
        hn = (p_b[0] + p_b[1]) / cnt_b[...]
        h2 = _leaky(
            jnp.dot(h1_b[...], wat[...], preferred_element_type=jnp.float32)
            + jnp.dot(hn, wab[...], preferred_element_type=jnp.float32)
            + b[...])
        ts_o[...] = jnp.dot(h2, ws[...], preferred_element_type=jnp.float32)
        td_o[...] = jnp.dot(h2, wd[...],
                            preferred_element_type=jnp.float32) + bb[...]

    return pl.pallas_call(
        body,
        grid=(N // _BN,),
        in_specs=[
            pl.BlockSpec((NC, _BN, D), lambda i: (0, i, 0)),
            pl.BlockSpec((_BN, D), lambda i: (i, 0)),
            pl.BlockSpec((_BN, D), lambda i: (i, 0)),
            pl.BlockSpec((D, D), lambda i: (0, 0)),
            pl.BlockSpec((D, D), lambda i: (0, 0)),
            pl.BlockSpec((1, D), lambda i: (0, 0)),
            pl.BlockSpec((D, D), lambda i: (0, 0)),
            pl.BlockSpec((D, D), lambda i: (0, 0)),
            pl.BlockSpec((1, D), lambda i: (0, 0)),
        ],
        out_specs=[
            pl.BlockSpec((_BN, D), lambda i: (i, 0)),
            pl.BlockSpec((_BN, D), lambda i: (i, 0)),
        ],
        out_shape=[
            jax.ShapeDtypeStruct((N, D), jnp.float32),
            jax.ShapeDtypeStruct((N, D), jnp.float32),
        ],
    )(p, cntb, h1, wa_t, wa_b, ba, wsrc, wdst, bcf)


def _final_heads(ps, scale, shift, wc2, bc2, wf2, bf2):
    def body(x_b, sc, sh, wc, bc, wf, bf, co_o, fo_o):
        x = x_b[...]
        xc = jnp.maximum(x[:, :32], 0.0)
        co_o[...] = jnp.dot(xc, wc[...],
                            preferred_element_type=jnp.float32) + bc[...]
        yf = jnp.maximum(x[:, 32:96] * sc[...] + sh[...], 0.0)
        fo_o[...] = jnp.dot(yf, wf[...],
                            preferred_element_type=jnp.float32) + bf[...]

    return pl.pallas_call(
        body,
        grid=(E // _BE,),
        in_specs=[
            pl.BlockSpec((_BE, D), lambda i: (i, 0)),
            pl.BlockSpec((1, 64), lambda i: (0, 0)),
            pl.BlockSpec((1, 64), lambda i: (0, 0)),
            pl.BlockSpec((32, 2), lambda i: (0, 0)),
            pl.BlockSpec((1, 2), lambda i: (0, 0)),
            pl.BlockSpec((64, NFINE), lambda i: (0, 0)),
            pl.BlockSpec((1, NFINE), lambda i: (0, 0)),
        ],
        out_specs=[
            pl.BlockSpec((_BE, 2), lambda i: (i, 0)),
            pl.BlockSpec((_BE, NFINE), lambda i: (i, 0)),
        ],
        out_shape=[
            jax.ShapeDtypeStruct((E, 2), jnp.float32),
            jax.ShapeDtypeStruct((E, NFINE), jnp.float32),
        ],
    )(ps, scale, shift, wc2, bc2, wf2, bf2)


def kernel(nfeats, efeats, edge_index, W_msg1, b_msg1, W_app1, b_app1,
           W_msg2, b_msg2, W_app2, b_app2, Wc1, bc1, Wc2, bc2,
           Wf1, bf1, bn_g, bn_b, Wf2, bf2):
    edges = edge_index.reshape(2 * E)
    efr = efeats.reshape(E * DE)

    q1 = _table1(nfeats, W_msg1[:D], b_msg1.reshape(1, D))
    p1, cnt = _msg_pass1(q1, edges, efr, W_msg1[D:])
    cntb = jnp.broadcast_to(
        jnp.maximum(cnt[:N] + cnt[NPAD:NPAD + N], 1.0).reshape(N, 1), (N, D))
    h1, q2 = _mid(p1, cntb, nfeats,
                  W_app1[:D], W_app1[D:], b_app1.reshape(1, D),
                  W_msg2[:HID], b_msg2.reshape(1, D))

    p2 = _msg_pass2(q2, edges, efr, W_msg2[HID:])
    zpad = jnp.zeros((D, 32), jnp.float32)
    wsrc = jnp.concatenate([Wc1[:D], Wf1[:D], zpad], axis=1)
    wdst = jnp.concatenate([Wc1[D:], Wf1[D:], zpad], axis=1)
    bcf = jnp.concatenate([bc1, bf1, jnp.zeros((32,), jnp.float32)])
    tsrc, tdst = _tail_tables(p2, cntb, h1,
                              W_app2[:HID], W_app2[HID:],
                              b_app2.reshape(1, D), wsrc, wdst,
                              bcf.reshape(1, D))

    ps, stats = _heads_pass(tsrc, tdst, edges)
    ssum = stats[:, 0:4, :].sum(axis=0).reshape(64)
    ssq = stats[:, 4:8, :].sum(axis=0).reshape(64)
    mean = ssum / E
    var = ssq / E - mean * mean
    scale = bn_g * lax.rsqrt(var + 1e-5)
    shift = bn_b - mean * scale
    coarse, fine = _final_heads(ps, scale.reshape(1, 64), shift.reshape(1, 64),
                                Wc2, bc2.reshape(1, 2), Wf2,
                                bf2.reshape(1, NFINE))
    return coarse, fine
